# Initial kernel scaffold; baseline (speedup 1.0000x reference)
import jax
import jax.numpy as jnp
from jax.experimental import pallas as pl


def kernel(x, hashed_weight):
    raise NotImplementedError("write your pallas kernel here")


# speedup vs baseline: 4996.4485x; 4996.4485x over previous
import functools

import jax
import jax.numpy as jnp
from jax import lax
from jax.experimental import pallas as pl
from jax.experimental.pallas import tpu as pltpu
from jax.experimental.pallas import tpu_sc as plsc

NUM_EMB = 100000
EMB_DIM = 64
HN = 3200000
ROWS = HN // EMB_DIM
BATCH = 4096
BAG = 20

NW = 32
BAGS_PER_W = BATCH // NW
CB = 4
CHUNK_IDX = CB * BAG
CHUNKS = BAGS_PER_W // CB
LANES = 16
COLS = EMB_DIM // LANES


def _bag_kernel(w2, idx):
    mesh = plsc.VectorSubcoreMesh(core_axis_name="c", subcore_axis_name="s")

    @functools.partial(
        pl.kernel,
        mesh=mesh,
        compiler_params=pltpu.CompilerParams(use_tc_tiling_on_sc=False),
        out_type=jax.ShapeDtypeStruct((BATCH, EMB_DIM), jnp.float32),
        scratch_types=[
            pltpu.VMEM((CHUNKS, CHUNK_IDX), jnp.int32),
            pltpu.VMEM((CHUNK_IDX, EMB_DIM), jnp.float32),
            pltpu.VMEM((BAGS_PER_W, EMB_DIM), jnp.float32),
            pltpu.SemaphoreType.DMA,
        ],
    )
    def k(w2_hbm, idx_hbm, out_hbm, idx_v, rows_v, out_v, sem):
        wid = lax.axis_index("s") * 2 + lax.axis_index("c")
        pltpu.sync_copy(idx_hbm.at[wid], idx_v)

        @pl.loop(0, CHUNKS)
        def _(c):
            for k5 in range(CHUNK_IDX // LANES):
                sl = pl.ds(k5 * LANES, LANES)
                v = idx_v[c, sl]
                idx_v[c, sl] = jnp.where(v >= ROWS, v - ROWS, v)
            pltpu.async_copy(w2_hbm.at[idx_v.at[c]], rows_v, sem).wait()
            for b in range(CB):
                for ch in range(COLS):
                    sl = pl.ds(ch * LANES, LANES)
                    acc = rows_v[b * BAG, sl]
                    for r in range(1, BAG):
                        acc = acc + rows_v[b * BAG + r, sl]
                    out_v[c * CB + b, sl] = acc

        pltpu.sync_copy(out_v, out_hbm.at[pl.ds(wid * BAGS_PER_W, BAGS_PER_W)])

    return k(w2, idx)


def kernel(x, hashed_weight):
    w2 = hashed_weight.reshape(ROWS, EMB_DIM)
    idx = x.reshape(NW, CHUNKS, CHUNK_IDX)
    return _bag_kernel(w2, idx)


# speedup vs baseline: 7013.7622x; 1.4037x over previous
import functools

import jax
import jax.numpy as jnp
from jax import lax
from jax.experimental import pallas as pl
from jax.experimental.pallas import tpu as pltpu
from jax.experimental.pallas import tpu_sc as plsc

NUM_EMB = 100000
EMB_DIM = 64
HN = 3200000
ROWS = HN // EMB_DIM
BATCH = 4096
BAG = 20

NW = 32
BAGS_PER_W = BATCH // NW
CB = 4
CHUNK_IDX = CB * BAG
CHUNKS = BAGS_PER_W // CB
LANES = 16
COLS = EMB_DIM // LANES


def _bag_kernel(w2, idx):
    mesh = plsc.VectorSubcoreMesh(core_axis_name="c", subcore_axis_name="s")

    @functools.partial(
        pl.kernel,
        mesh=mesh,
        compiler_params=pltpu.CompilerParams(use_tc_tiling_on_sc=False),
        out_type=jax.ShapeDtypeStruct((BATCH, EMB_DIM), jnp.float32),
        scratch_types=[
            pltpu.VMEM((CHUNKS, CHUNK_IDX), jnp.int32),
            pltpu.VMEM((CHUNK_IDX, EMB_DIM), jnp.float32),
            pltpu.VMEM((CHUNK_IDX, EMB_DIM), jnp.float32),
            pltpu.VMEM((BAGS_PER_W, EMB_DIM), jnp.float32),
            pltpu.SemaphoreType.DMA,
            pltpu.SemaphoreType.DMA,
        ],
    )
    def k(w2_hbm, idx_hbm, out_hbm, idx_v, rows0, rows1, out_v, sem0, sem1):
        wid = lax.axis_index("s") * 2 + lax.axis_index("c")
        pltpu.sync_copy(idx_hbm.at[wid], idx_v)

        @pl.loop(0, CHUNKS)
        def _(c):
            for k5 in range(CHUNK_IDX // LANES):
                sl = pl.ds(k5 * LANES, LANES)
                v = idx_v[c, sl]
                idx_v[c, sl] = jnp.where(v >= ROWS, v - ROWS, v)

        def start(c, buf, sem):
            pltpu.async_copy(w2_hbm.at[idx_v.at[c]], buf, sem)

        def wait(c, buf, sem):
            pltpu.make_async_copy(w2_hbm.at[idx_v.at[c]], buf, sem).wait()

        def accum(c, buf):
            for b in range(CB):
                for ch in range(COLS):
                    sl = pl.ds(ch * LANES, LANES)
                    vals = [buf[b * BAG + r, sl] for r in range(BAG)]
                    while len(vals) > 1:
                        nxt = [vals[i] + vals[i + 1]
                               for i in range(0, len(vals) - 1, 2)]
                        if len(vals) % 2:
                            nxt.append(vals[-1])
                        vals = nxt
                    out_v[c * CB + b, sl] = vals[0]

        start(0, rows0, sem0)

        @pl.loop(0, CHUNKS - 2, step=2)
        def _(c):
            start(c + 1, rows1, sem1)
            wait(c, rows0, sem0)
            accum(c, rows0)
            start(c + 2, rows0, sem0)
            wait(c + 1, rows1, sem1)
            accum(c + 1, rows1)

        start(CHUNKS - 1, rows1, sem1)
        wait(CHUNKS - 2, rows0, sem0)
        accum(CHUNKS - 2, rows0)
        wait(CHUNKS - 1, rows1, sem1)
        accum(CHUNKS - 1, rows1)

        pltpu.sync_copy(out_v, out_hbm.at[pl.ds(wid * BAGS_PER_W, BAGS_PER_W)])

    return k(w2, idx)


def kernel(x, hashed_weight):
    w2 = hashed_weight.reshape(ROWS, EMB_DIM)
    idx = x.reshape(NW, CHUNKS, CHUNK_IDX)
    return _bag_kernel(w2, idx)
